# baseline (speedup 1.0000x reference)
import functools

import jax
import jax.numpy as jnp
from jax import lax
from jax.experimental import pallas as pl
from jax.experimental.pallas import tpu as pltpu
from jax.experimental.pallas import tpu_sc as plsc

BATCH = 16384
EMBED = 16
HIDDEN = 128
_GROUP = 128 // EMBED

_NC = 2
_NS = 16
_NW = _NC * _NS
_BPW = BATCH // _NW
_L = 16


@functools.cache
def _make_sc_gather():
    mesh = plsc.VectorSubcoreMesh(core_axis_name="c", subcore_axis_name="s")

    @functools.partial(
        pl.kernel,
        mesh=mesh,
        out_type=jax.ShapeDtypeStruct((EMBED, BATCH), jnp.float32),
        scratch_types=[
            pltpu.VMEM((_BPW,), jnp.int32),
            pltpu.VMEM((_BPW,), jnp.int32),
            pltpu.VMEM((_BPW, 128), jnp.float32),
            pltpu.VMEM((EMBED, _BPW), jnp.float32),
            pltpu.SemaphoreType.DMA,
        ],
        compiler_params=pltpu.CompilerParams(needs_layout_passes=False),
    )
    def _sc_gather(packed_hbm, idx_hbm, out_hbm, idx_v, gidx_v, rows_v,
                   et_v, sem):
        wid = lax.axis_index("s") * _NC + lax.axis_index("c")
        base = wid * _BPW
        pltpu.sync_copy(idx_hbm.at[pl.ds(base, _BPW)], idx_v)

        def to_group(g, carry):
            v = idx_v[pl.ds(g * _L, _L)]
            gidx_v[pl.ds(g * _L, _L)] = v >> 3
            return carry

        lax.fori_loop(0, _BPW // _L, to_group, 0)
        pltpu.async_copy(packed_hbm.at[gidx_v], rows_v, sem).wait()

        lanes = lax.iota(jnp.int32, _L)

        def extract(g, carry):
            v = idx_v[pl.ds(g * _L, _L)]
            off = (v & 7) << 4
            rows = lanes + g * _L
            for c in range(EMBED):
                vals = plsc.load_gather(rows_v, [rows, off + c])
                et_v[c, pl.ds(g * _L, _L)] = vals
            return carry

        lax.fori_loop(0, _BPW // _L, extract, 0)
        pltpu.sync_copy(et_v, out_hbm.at[:, pl.ds(base, _BPW)])

    return _sc_gather


def _dense_body(et_ref, w_ref, b_ref, g_ref, beta_ref, o_ref):
    et = et_ref[...]
    w = w_ref[...]
    y = lax.dot_general(
        et, w, (((0,), (1,)), ((), ())),
        preferred_element_type=jnp.float32,
    ) + b_ref[...]
    mean = jnp.mean(y, axis=0, keepdims=True)
    var = jnp.mean(jnp.square(y - mean), axis=0, keepdims=True)
    yn = (y - mean) * lax.rsqrt(var + 1e-5)
    yn = yn * g_ref[...] + beta_ref[...]
    sp = jnp.log1p(jnp.exp(-jnp.abs(yn))) + jnp.maximum(yn, 0.0)
    o_ref[...] = yn * jnp.tanh(sp)


@jax.jit
def _dense(et, w, b, g, beta):
    return pl.pallas_call(
        _dense_body,
        out_shape=jax.ShapeDtypeStruct((BATCH, HIDDEN), jnp.float32),
    )(et, w, b.reshape(1, HIDDEN), g.reshape(1, HIDDEN),
      beta.reshape(1, HIDDEN))


def kernel(x, table, W, b, gamma, beta):
    xi = x.astype(jnp.int32)
    packed = table.reshape(table.shape[0] // _GROUP, 128)
    et = _make_sc_gather()(packed, xi)
    return _dense(et, W, b, gamma, beta)


# speedup vs baseline: 4.2671x; 4.2671x over previous
import functools

import jax
import jax.numpy as jnp
from jax import lax
from jax.experimental import pallas as pl
from jax.experimental.pallas import tpu as pltpu
from jax.experimental.pallas import tpu_sc as plsc

BATCH = 16384
EMBED = 16
HIDDEN = 128

_NC = 2
_NS = 16
_NW = _NC * _NS
_BPW = BATCH // _NW
_L = 16
_CH = 16


@functools.cache
def _make_sc_gather():
    mesh = plsc.VectorSubcoreMesh(core_axis_name="c", subcore_axis_name="s")

    @functools.partial(
        pl.kernel,
        mesh=mesh,
        out_type=jax.ShapeDtypeStruct((EMBED, BATCH), jnp.float32),
        scratch_types=[
            pltpu.VMEM((_BPW,), jnp.int32),
            pltpu.VMEM((_CH * EMBED, 128), jnp.float32),
            pltpu.VMEM((EMBED, _BPW), jnp.float32),
            pltpu.SemaphoreType.DMA,
        ],
        compiler_params=pltpu.CompilerParams(needs_layout_passes=False),
    )
    def _sc_gather(table_t_hbm, idx_hbm, out_hbm, idx_v, stage_v, et_v, sem):
        wid = lax.axis_index("s") * _NC + lax.axis_index("c")
        base = wid * _BPW
        pltpu.sync_copy(idx_hbm.at[pl.ds(base, _BPW)], idx_v)

        lanes = lax.iota(jnp.int32, _L)

        def chunk(g, carry):
            v = idx_v[pl.ds(g * _CH, _CH)]
            rq = (v >> 7) << 7
            rl = v & 127
            copies = []
            for k in range(_CH):
                c = pltpu.async_copy(
                    table_t_hbm.at[:, pl.ds(pl.multiple_of(rq[k], 128), 128)],
                    stage_v.at[pl.ds(k * EMBED, EMBED)],
                    sem,
                )
                copies.append(c)
            for c in copies:
                c.wait()
            for k in range(_CH):
                rows = lanes + k * EMBED
                cols = jnp.full((_L,), rl[k], jnp.int32)
                vals = plsc.load_gather(stage_v, [rows, cols])
                plsc.store_scatter(
                    et_v, [lanes, jnp.full((_L,), g * _CH + k, jnp.int32)], vals
                )
            return carry

        lax.fori_loop(0, _BPW // _CH, chunk, 0)
        pltpu.sync_copy(et_v, out_hbm.at[:, pl.ds(base, _BPW)])

    return _sc_gather


def _dense_body(et_ref, w_ref, b_ref, g_ref, beta_ref, o_ref):
    et = et_ref[...]
    w = w_ref[...]
    y = lax.dot_general(
        et, w, (((0,), (1,)), ((), ())),
        preferred_element_type=jnp.float32,
    ) + b_ref[...]
    mean = jnp.mean(y, axis=0, keepdims=True)
    var = jnp.mean(jnp.square(y - mean), axis=0, keepdims=True)
    yn = (y - mean) * lax.rsqrt(var + 1e-5)
    yn = yn * g_ref[...] + beta_ref[...]
    sp = jnp.log1p(jnp.exp(-jnp.abs(yn))) + jnp.maximum(yn, 0.0)
    o_ref[...] = yn * jnp.tanh(sp)


@jax.jit
def _dense(et, w, b, g, beta):
    return pl.pallas_call(
        _dense_body,
        out_shape=jax.ShapeDtypeStruct((BATCH, HIDDEN), jnp.float32),
    )(et, w, b.reshape(1, HIDDEN), g.reshape(1, HIDDEN),
      beta.reshape(1, HIDDEN))


def kernel(x, table, W, b, gamma, beta):
    xi = x.astype(jnp.int32)
    et = _make_sc_gather()(table.T, xi)
    return _dense(et, W, b, gamma, beta)


# speedup vs baseline: 5.1210x; 1.2001x over previous
import functools

import jax
import jax.numpy as jnp
from jax import lax
from jax.experimental import pallas as pl
from jax.experimental.pallas import tpu as pltpu
from jax.experimental.pallas import tpu_sc as plsc

BATCH = 16384
EMBED = 16
HIDDEN = 128

_NC = 2
_NS = 16
_NW = _NC * _NS
_BPW = BATCH // _NW
_L = 16
_CH = 16


@functools.cache
def _make_sc_gather():
    mesh = plsc.VectorSubcoreMesh(core_axis_name="c", subcore_axis_name="s")

    @functools.partial(
        pl.kernel,
        mesh=mesh,
        out_type=jax.ShapeDtypeStruct((EMBED, BATCH), jnp.float32),
        scratch_types=[
            pltpu.VMEM((_BPW,), jnp.int32),
            pltpu.VMEM((_CH * EMBED, 128), jnp.float32),
            pltpu.VMEM((_CH * EMBED, 128), jnp.float32),
            pltpu.VMEM((EMBED, _BPW), jnp.float32),
            pltpu.SemaphoreType.DMA,
            pltpu.SemaphoreType.DMA,
        ],
        compiler_params=pltpu.CompilerParams(needs_layout_passes=False),
    )
    def _sc_gather(table_t_hbm, idx_hbm, out_hbm, idx_v, stage0_v, stage1_v,
                   et_v, sem0, sem1):
        wid = lax.axis_index("s") * _NC + lax.axis_index("c")
        base = wid * _BPW
        pltpu.sync_copy(idx_hbm.at[pl.ds(base, _BPW)], idx_v)

        lanes = lax.iota(jnp.int32, _L)
        n_chunks = _BPW // _CH

        def issue(g, stage_v, sem):
            v = idx_v[pl.ds(g * _CH, _CH)]
            rq = (v >> 7) << 7
            for k in range(_CH):
                pltpu.async_copy(
                    table_t_hbm.at[:, pl.ds(pl.multiple_of(rq[k], 128), 128)],
                    stage_v.at[pl.ds(k * EMBED, EMBED)],
                    sem,
                )

        def drain_and_extract(g, stage_v, sem):
            for k in range(_CH):
                pltpu.make_async_copy(
                    table_t_hbm.at[:, pl.ds(0, 128)],
                    stage_v.at[pl.ds(k * EMBED, EMBED)],
                    sem,
                ).wait()
            v = idx_v[pl.ds(g * _CH, _CH)]
            rl = v & 127
            for k in range(_CH):
                rows = lanes + k * EMBED
                cols = jnp.full((_L,), rl[k], jnp.int32)
                vals = plsc.load_gather(stage_v, [rows, cols])
                plsc.store_scatter(
                    et_v, [lanes, jnp.full((_L,), g * _CH + k, jnp.int32)], vals
                )

        issue(0, stage0_v, sem0)

        def pair(t, carry):
            issue(2 * t + 1, stage1_v, sem1)
            drain_and_extract(2 * t, stage0_v, sem0)

            @pl.when(t + 1 < n_chunks // 2)
            def _():
                issue(2 * t + 2, stage0_v, sem0)

            drain_and_extract(2 * t + 1, stage1_v, sem1)
            return carry

        lax.fori_loop(0, n_chunks // 2, pair, 0)
        pltpu.sync_copy(et_v, out_hbm.at[:, pl.ds(base, _BPW)])

    return _sc_gather


def _dense_body(et_ref, w_ref, b_ref, g_ref, beta_ref, o_ref):
    et = et_ref[...]
    w = w_ref[...]
    y = lax.dot_general(
        et, w, (((0,), (1,)), ((), ())),
        preferred_element_type=jnp.float32,
    ) + b_ref[...]
    mean = jnp.mean(y, axis=0, keepdims=True)
    var = jnp.mean(jnp.square(y - mean), axis=0, keepdims=True)
    yn = (y - mean) * lax.rsqrt(var + 1e-5)
    yn = yn * g_ref[...] + beta_ref[...]
    u = jnp.exp(jnp.minimum(yn, 20.0))
    a = u * (u + 2.0)
    o_ref[...] = yn * (a / (a + 2.0))


@jax.jit
def _dense(et, w, b, g, beta):
    return pl.pallas_call(
        _dense_body,
        out_shape=jax.ShapeDtypeStruct((BATCH, HIDDEN), jnp.float32),
    )(et, w, b.reshape(1, HIDDEN), g.reshape(1, HIDDEN),
      beta.reshape(1, HIDDEN))


def kernel(x, table, W, b, gamma, beta):
    xi = x.astype(jnp.int32)
    et = _make_sc_gather()(table.T, xi)
    return _dense(et, W, b, gamma, beta)


# speedup vs baseline: 5.4863x; 1.0713x over previous
import functools

import jax
import jax.numpy as jnp
from jax import lax
from jax.experimental import pallas as pl
from jax.experimental.pallas import tpu as pltpu
from jax.experimental.pallas import tpu_sc as plsc

BATCH = 16384
EMBED = 16
HIDDEN = 128

_NC = 2
_NS = 16
_NW = _NC * _NS
_BPW = BATCH // _NW
_L = 16
_CH = 16


@functools.cache
def _make_sc_gather():
    mesh = plsc.VectorSubcoreMesh(core_axis_name="c", subcore_axis_name="s")

    @functools.partial(
        pl.kernel,
        mesh=mesh,
        out_type=jax.ShapeDtypeStruct((EMBED, BATCH), jnp.float32),
        scratch_types=[
            pltpu.VMEM((_BPW,), jnp.int32),
            pltpu.VMEM((_CH * EMBED, 128), jnp.float32),
            pltpu.VMEM((_CH * EMBED, 128), jnp.float32),
            pltpu.VMEM((_CH * EMBED, 128), jnp.float32),
            pltpu.VMEM((EMBED, _BPW), jnp.float32),
            pltpu.SemaphoreType.DMA,
            pltpu.SemaphoreType.DMA,
            pltpu.SemaphoreType.DMA,
        ],
        compiler_params=pltpu.CompilerParams(needs_layout_passes=False),
    )
    def _sc_gather(table_t_hbm, idx_hbm, out_hbm, idx_v, stage0_v, stage1_v,
                   stage2_v, et_v, sem0, sem1, sem2):
        wid = lax.axis_index("s") * _NC + lax.axis_index("c")
        base = wid * _BPW
        pltpu.sync_copy(idx_hbm.at[pl.ds(base, _BPW)], idx_v)

        lanes = lax.iota(jnp.int32, _L)
        n_chunks = _BPW // _CH

        def issue(g, stage_v, sem):
            v = idx_v[pl.ds(g * _CH, _CH)]
            rq = (v >> 7) << 7
            for k in range(_CH):
                pltpu.async_copy(
                    table_t_hbm.at[:, pl.ds(pl.multiple_of(rq[k], 128), 128)],
                    stage_v.at[pl.ds(k * EMBED, EMBED)],
                    sem,
                )

        def drain_and_extract(g, stage_v, sem):
            for k in range(_CH):
                pltpu.make_async_copy(
                    table_t_hbm.at[:, pl.ds(0, 128)],
                    stage_v.at[pl.ds(k * EMBED, EMBED)],
                    sem,
                ).wait()
            v = idx_v[pl.ds(g * _CH, _CH)]
            rl = v & 127
            for k in range(_CH):
                rows = lanes + k * EMBED
                cols = jnp.full((_L,), rl[k], jnp.int32)
                vals = plsc.load_gather(stage_v, [rows, cols])
                plsc.store_scatter(
                    et_v, [lanes, jnp.full((_L,), g * _CH + k, jnp.int32)], vals
                )

        bufs = ((stage0_v, sem0), (stage1_v, sem1), (stage2_v, sem2))
        issue(0, stage0_v, sem0)
        issue(1, stage1_v, sem1)

        def triple(t, carry):
            for p in range(3):
                g = 3 * t + p
                nxt = bufs[(p + 2) % 3]

                @pl.when(g + 2 < n_chunks)
                def _():
                    issue(g + 2, nxt[0], nxt[1])

                @pl.when(g < n_chunks)
                def _():
                    drain_and_extract(g, bufs[p][0], bufs[p][1])

            return carry

        lax.fori_loop(0, pl.cdiv(n_chunks, 3), triple, 0)
        pltpu.sync_copy(et_v, out_hbm.at[:, pl.ds(base, _BPW)])

    return _sc_gather


def _dense_body(et_ref, w_ref, b_ref, g_ref, beta_ref, o_ref):
    et = et_ref[...]
    w = w_ref[...]
    y = lax.dot_general(
        et, w, (((0,), (1,)), ((), ())),
        preferred_element_type=jnp.float32,
    ) + b_ref[...]
    mean = jnp.mean(y, axis=0, keepdims=True)
    var = jnp.mean(jnp.square(y - mean), axis=0, keepdims=True)
    yn = (y - mean) * lax.rsqrt(var + 1e-5)
    yn = yn * g_ref[...] + beta_ref[...]
    u = jnp.exp(jnp.minimum(yn, 20.0))
    a = u * (u + 2.0)
    o_ref[...] = yn * (a / (a + 2.0))


@jax.jit
def _dense(et, w, b, g, beta):
    return pl.pallas_call(
        _dense_body,
        out_shape=jax.ShapeDtypeStruct((BATCH, HIDDEN), jnp.float32),
    )(et, w, b.reshape(1, HIDDEN), g.reshape(1, HIDDEN),
      beta.reshape(1, HIDDEN))


def kernel(x, table, W, b, gamma, beta):
    xi = x.astype(jnp.int32)
    et = _make_sc_gather()(table.T, xi)
    return _dense(et, W, b, gamma, beta)


# speedup vs baseline: 5.5539x; 1.0123x over previous
import functools

import jax
import jax.numpy as jnp
from jax import lax
from jax.experimental import pallas as pl
from jax.experimental.pallas import tpu as pltpu
from jax.experimental.pallas import tpu_sc as plsc

BATCH = 16384
EMBED = 16
HIDDEN = 128

_NC = 2
_NS = 16
_NW = _NC * _NS
_BPW = BATCH // _NW
_L = 16
_CH = 16


@functools.cache
def _make_sc_gather():
    mesh = plsc.VectorSubcoreMesh(core_axis_name="c", subcore_axis_name="s")

    @functools.partial(
        pl.kernel,
        mesh=mesh,
        out_type=jax.ShapeDtypeStruct((EMBED, BATCH), jnp.float32),
        scratch_types=[
            pltpu.VMEM((_BPW,), jnp.int32),
            pltpu.VMEM((_CH * EMBED, 128), jnp.float32),
            pltpu.VMEM((_CH * EMBED, 128), jnp.float32),
            pltpu.VMEM((_CH * EMBED, 128), jnp.float32),
            pltpu.VMEM((EMBED, _BPW), jnp.float32),
            pltpu.SemaphoreType.DMA,
            pltpu.SemaphoreType.DMA,
            pltpu.SemaphoreType.DMA,
        ],
        compiler_params=pltpu.CompilerParams(needs_layout_passes=False),
    )
    def _sc_gather(table_t_hbm, idx_hbm, out_hbm, idx_v, stage0_v, stage1_v,
                   stage2_v, et_v, sem0, sem1, sem2):
        wid = lax.axis_index("s") * _NC + lax.axis_index("c")
        base = wid * _BPW
        pltpu.sync_copy(idx_hbm.at[pl.ds(base, _BPW)], idx_v)

        lanes = lax.iota(jnp.int32, _L)
        n_chunks = _BPW // _CH

        def issue(g, stage_v, sem):
            v = idx_v[pl.ds(g * _CH, _CH)]
            rq = (v >> 7) << 7
            for k in range(_CH):
                pltpu.async_copy(
                    table_t_hbm.at[:, pl.ds(pl.multiple_of(rq[k], 128), 128)],
                    stage_v.at[pl.ds(k * EMBED, EMBED)],
                    sem,
                )

        def drain_and_extract(g, stage_v, sem):
            for k in range(_CH):
                pltpu.make_async_copy(
                    table_t_hbm.at[:, pl.ds(0, 128)],
                    stage_v.at[pl.ds(k * EMBED, EMBED)],
                    sem,
                ).wait()
            v = idx_v[pl.ds(g * _CH, _CH)]
            rl = v & 127
            for k in range(_CH):
                rows = lanes + k * EMBED
                cols = jnp.full((_L,), rl[k], jnp.int32)
                vals = plsc.load_gather(stage_v, [rows, cols])
                plsc.store_scatter(
                    et_v, [lanes, jnp.full((_L,), g * _CH + k, jnp.int32)], vals
                )

        bufs = ((stage0_v, sem0), (stage1_v, sem1), (stage2_v, sem2))
        issue(0, stage0_v, sem0)
        issue(1, stage1_v, sem1)

        def triple(t, carry):
            for p in range(3):
                g = 3 * t + p
                nxt = bufs[(p + 2) % 3]

                @pl.when(g + 2 < n_chunks)
                def _():
                    issue(g + 2, nxt[0], nxt[1])

                @pl.when(g < n_chunks)
                def _():
                    drain_and_extract(g, bufs[p][0], bufs[p][1])

            return carry

        lax.fori_loop(0, pl.cdiv(n_chunks, 3), triple, 0)
        pltpu.sync_copy(et_v, out_hbm.at[:, pl.ds(base, _BPW)])

    return _sc_gather


_NB = 8
_BB = BATCH // _NB


def _dense_body(et_ref, w_ref, b_ref, g_ref, beta_ref, o_ref, y_v, s_v):
    i = pl.program_id(0)
    phase0 = i < _NB

    @pl.when(phase0)
    def _():
        blk = et_ref[...]
        y = lax.dot_general(
            blk, w_ref[...], (((0,), (1,)), ((), ())),
            preferred_element_type=jnp.float32,
        ) + b_ref[...]
        y_v[pl.ds(i * _BB, _BB), :] = y

        @pl.when(i == 0)
        def _():
            s_v[...] = jnp.zeros_like(s_v)

        s_v[0:1, :] += jnp.sum(y, axis=0, keepdims=True)
        s_v[1:2, :] += jnp.sum(y * y, axis=0, keepdims=True)

    @pl.when(jnp.logical_not(phase0))
    def _():
        j = i - _NB
        mean = s_v[0:1, :] * (1.0 / BATCH)
        var = s_v[1:2, :] * (1.0 / BATCH) - mean * mean
        y = y_v[pl.ds(j * _BB, _BB), :]
        yn = (y - mean) * lax.rsqrt(var + 1e-5)
        yn = yn * g_ref[...] + beta_ref[...]
        u = jnp.exp(jnp.minimum(yn, 20.0))
        a = u * (u + 2.0)
        o_ref[...] = yn * (a / (a + 2.0))


@jax.jit
def _dense(et, w, b, g, beta):
    return pl.pallas_call(
        _dense_body,
        grid=(2 * _NB,),
        in_specs=[
            pl.BlockSpec((EMBED, _BB), lambda i: (0, jnp.minimum(i, _NB - 1))),
            pl.BlockSpec((HIDDEN, EMBED), lambda i: (0, 0)),
            pl.BlockSpec((1, HIDDEN), lambda i: (0, 0)),
            pl.BlockSpec((1, HIDDEN), lambda i: (0, 0)),
            pl.BlockSpec((1, HIDDEN), lambda i: (0, 0)),
        ],
        out_specs=pl.BlockSpec(
            (_BB, HIDDEN), lambda i: (jnp.maximum(i - _NB, 0), 0)
        ),
        out_shape=jax.ShapeDtypeStruct((BATCH, HIDDEN), jnp.float32),
        scratch_shapes=[
            pltpu.VMEM((BATCH, HIDDEN), jnp.float32),
            pltpu.VMEM((2, HIDDEN), jnp.float32),
        ],
    )(et, w, b.reshape(1, HIDDEN), g.reshape(1, HIDDEN),
      beta.reshape(1, HIDDEN))


def kernel(x, table, W, b, gamma, beta):
    xi = x.astype(jnp.int32)
    et = _make_sc_gather()(table.T, xi)
    return _dense(et, W, b, gamma, beta)
